# Initial kernel scaffold; baseline (speedup 1.0000x reference)
import jax
import jax.numpy as jnp
from jax.experimental import pallas as pl


def kernel(q, k, embeddings_table):
    raise NotImplementedError("write your pallas kernel here")


# speedup vs baseline: 2.5243x; 2.5243x over previous
import functools

import jax
import jax.numpy as jnp
from jax.experimental import pallas as pl
from jax.experimental.pallas import tpu as pltpu

_MAX_POSITION = 512
_CHUNK = 256


def _rpe_kernel(table_ref, out_ref, e2_ref, *, len_q, len_k, hidden, p):
    i = pl.program_id(0)
    c = pl.program_id(1)
    base = ((p - len_q) // 8) * 8
    ncopy = ((2 * p - base) // 8) * 8
    e2_rows = e2_ref.shape[0]

    @pl.when(jnp.logical_and(i == 0, c == 0))
    def _build_e2():
        e2_ref[...] = jnp.broadcast_to(
            table_ref[2 * p : 2 * p + 1, :], (e2_rows, hidden)
        )
        e2_ref[0:ncopy, :] = table_ref[base : base + ncopy, :]

    shift = (p - base) - i
    s8 = pl.multiple_of((shift // 8) * 8, 8)
    r = shift % 8
    win = _CHUNK + 8
    a = e2_ref[pl.ds(s8 + c * _CHUNK, win), :]
    rolled = pltpu.roll(a, (-r) % win, axis=0)
    out_ref[0, :, :] = rolled[0:_CHUNK, :]


def kernel(q, k, embeddings_table):
    len_q = q.shape[1]
    len_k = k.shape[1]
    hidden = embeddings_table.shape[1]
    p = _MAX_POSITION
    n_chunks = len_k // _CHUNK
    max_shift = p - ((p - len_q) // 8) * 8
    e2_rows = ((max_shift + 7) // 8) * 8 + len_k + 8

    body = functools.partial(
        _rpe_kernel, len_q=len_q, len_k=len_k, hidden=hidden, p=p
    )
    return pl.pallas_call(
        body,
        grid=(len_q, n_chunks),
        in_specs=[
            pl.BlockSpec(embeddings_table.shape, lambda i, c: (0, 0)),
        ],
        out_specs=pl.BlockSpec((1, _CHUNK, hidden), lambda i, c: (i, c, 0)),
        out_shape=jax.ShapeDtypeStruct((len_q, len_k, hidden), jnp.float32),
        scratch_shapes=[pltpu.VMEM((e2_rows, hidden), jnp.float32)],
        compiler_params=pltpu.CompilerParams(
            dimension_semantics=("arbitrary", "arbitrary"),
        ),
    )(embeddings_table)
